# baseline (speedup 1.0000x reference)
import functools

import jax
import jax.numpy as jnp
from jax import lax
from jax.experimental import pallas as pl
from jax.experimental.pallas import tpu as pltpu
from jax.experimental.pallas import tpu_sc as plsc

VOCAB = 100000
EMB = 300
EPAD = 320
NW32 = EPAD // 2
NBLK = EPAD // 32
HID = 100
NCLS = 1000
B = 4096
L = 200
HALF = L // 2

NC, NS = 2, 16
NW = NC * NS
BPW = B // NW


def _make_pool_kernel():
    mesh = plsc.VectorSubcoreMesh(core_axis_name="c", subcore_axis_name="s")

    @functools.partial(
        pl.kernel,
        mesh=mesh,
        out_type=jax.ShapeDtypeStruct((B, EPAD), jnp.float32),
        compiler_params=pltpu.CompilerParams(
            use_tc_tiling_on_sc=False, needs_layout_passes=False
        ),
        scratch_types=[
            pltpu.VMEM((BPW, 2, HALF), jnp.int32),
            pltpu.VMEM((HALF, NW32), jnp.int32),
            pltpu.VMEM((HALF, NW32), jnp.int32),
            pltpu.VMEM((HALF, NW32), jnp.int32),
            pltpu.VMEM((HALF, NW32), jnp.int32),
            pltpu.VMEM((EPAD,), jnp.float32),
            pltpu.VMEM((EPAD,), jnp.float32),
            pltpu.SemaphoreType.DMA,
            pltpu.SemaphoreType.DMA,
            pltpu.SemaphoreType.DMA,
            pltpu.SemaphoreType.DMA,
            pltpu.SemaphoreType.DMA,
            pltpu.SemaphoreType.DMA,
        ],
    )
    def pool(idx_hbm, tbl_hbm, out_hbm, idx_v, ga0, ga1, gb0, gb1,
             orow_a, orow_b, sa0, sa1, sb0, sb1, soa, sob):
        wid = lax.axis_index("s") * NC + lax.axis_index("c")
        base = wid * BPW
        pltpu.sync_copy(idx_hbm.at[pl.ds(base, BPW)], idx_v)

        def fire(b, g0, g1, s0, s1):
            pltpu.async_copy(tbl_hbm.at[idx_v.at[b, 0]], g0, s0)
            pltpu.async_copy(tbl_hbm.at[idx_v.at[b, 1]], g1, s1)

        def drain(b, g0, g1, s0, s1):
            pltpu.make_async_copy(tbl_hbm.at[idx_v.at[b, 0]], g0, s0).wait()
            pltpu.make_async_copy(tbl_hbm.at[idx_v.at[b, 1]], g1, s1).wait()

        def sum_rows(gbuf, acc):
            def row_body(r, a):
                out = []
                for j in range(NBLK):
                    lo, hi = plsc.unpack(
                        plsc.bitcast(gbuf[r, pl.ds(16 * j, 16)], jnp.bfloat16),
                        format=plsc.PackFormat.INTERLEAVED,
                    )
                    out.append(a[2 * j] + lo)
                    out.append(a[2 * j + 1] + hi)
                return tuple(out)
            return lax.fori_loop(0, HALF, row_body, acc, unroll=4)

        def process(b, g0, g1, s0, s1, orow, so):
            drain(b, g0, g1, s0, s1)
            acc = tuple(jnp.zeros((16,), jnp.float32) for _ in range(2 * NBLK))
            acc = sum_rows(g0, acc)
            acc = sum_rows(g1, acc)
            pltpu.make_async_copy(orow, out_hbm.at[base + b], so).wait()
            for j in range(NBLK):
                orow[pl.ds(16 * j, 16)] = acc[2 * j]
                orow[pl.ds(160 + 16 * j, 16)] = acc[2 * j + 1]
            pltpu.async_copy(orow, out_hbm.at[base + b], so)

        fire(0, ga0, ga1, sa0, sa1)
        pltpu.async_copy(orow_a, out_hbm.at[base], soa)
        pltpu.async_copy(orow_b, out_hbm.at[base + 1], sob)

        def pair_body(g, carry):
            b0 = 2 * g
            fire(b0 + 1, gb0, gb1, sb0, sb1)
            process(b0, ga0, ga1, sa0, sa1, orow_a, soa)
            b2 = jnp.minimum(b0 + 2, BPW - 1)
            fire(b2, ga0, ga1, sa0, sa1)
            process(b0 + 1, gb0, gb1, sb0, sb1, orow_b, sob)
            return carry

        lax.fori_loop(0, BPW // 2, pair_body, 0)

        drain(BPW - 1, ga0, ga1, sa0, sa1)
        pltpu.make_async_copy(orow_a, out_hbm.at[base + BPW - 2], soa).wait()
        pltpu.make_async_copy(orow_b, out_hbm.at[base + BPW - 1], sob).wait()

    return pool


_pool = _make_pool_kernel()

PADBLK = 10000
NHI = EMB - NW32


def _cvt_body(src_ref, dst_ref):
    u = jax.lax.bitcast_convert_type(src_ref[...], jnp.uint32)
    r = (u + 0x7FFF + ((u >> 16) & 1)) & jnp.uint32(0xFFFF0000)
    lo = r[:, :NW32] >> 16
    hi = r[:, NW32:]
    dst_ref[:, :NHI] = jax.lax.bitcast_convert_type(
        lo[:, :NHI] | hi, jnp.int32
    )
    dst_ref[:, NHI:] = jax.lax.bitcast_convert_type(lo[:, NHI:], jnp.int32)


def _cvt_table(tbl):
    return pl.pallas_call(
        _cvt_body,
        grid=(VOCAB // PADBLK,),
        in_specs=[pl.BlockSpec((PADBLK, EMB), lambda i: (i, 0))],
        out_specs=pl.BlockSpec((PADBLK, NW32), lambda i: (i, 0)),
        out_shape=jax.ShapeDtypeStruct((VOCAB, NW32), jnp.int32),
    )(tbl)


BLK = 512


def _mlp_body(enc_ref, tl_ref, w1_ref, b1_ref, w2_ref, b2_ref, out_ref):
    x = enc_ref[...] / tl_ref[...]
    h = jnp.dot(x, w1_ref[...], preferred_element_type=jnp.float32) + b1_ref[...]
    h = jnp.where(h > 0, h, jnp.exp(h) - 1.0)
    out_ref[...] = (
        jnp.dot(h, w2_ref[...], preferred_element_type=jnp.float32) + b2_ref[...]
    )


def kernel(input_text, text_len, emb_table, W1, b1, W2, b2):
    idx3 = input_text.astype(jnp.int32).reshape(B, 2, HALF)
    w1t = jnp.pad(W1, ((0, 0), (0, EPAD - EMB))).T
    w2t = W2.T
    b1r = b1.reshape(1, HID)
    b2r = b2.reshape(1, NCLS)
    tl2 = text_len.reshape(B, 1)

    tbl = _cvt_table(emb_table)
    encoded = _pool(idx3, tbl)

    logits = pl.pallas_call(
        _mlp_body,
        grid=(B // BLK,),
        in_specs=[
            pl.BlockSpec((BLK, EPAD), lambda i: (i, 0)),
            pl.BlockSpec((BLK, 1), lambda i: (i, 0)),
            pl.BlockSpec((EPAD, HID), lambda i: (0, 0)),
            pl.BlockSpec((1, HID), lambda i: (0, 0)),
            pl.BlockSpec((HID, NCLS), lambda i: (0, 0)),
            pl.BlockSpec((1, NCLS), lambda i: (0, 0)),
        ],
        out_specs=pl.BlockSpec((BLK, NCLS), lambda i: (i, 0)),
        out_shape=jax.ShapeDtypeStruct((B, NCLS), jnp.float32),
    )(encoded, tl2, w1t, b1r, w2t, b2r)
    return logits


# speedup vs baseline: 1.0358x; 1.0358x over previous
import functools

import jax
import jax.numpy as jnp
from jax import lax
from jax.experimental import pallas as pl
from jax.experimental.pallas import tpu as pltpu
from jax.experimental.pallas import tpu_sc as plsc

VOCAB = 100000
EMB = 300
EPAD = 320
NW32 = EPAD // 2
NBLK = EPAD // 32
HID = 100
NCLS = 1000
B = 4096
L = 200
HALF = L // 2

NC, NS = 2, 16
NW = NC * NS
BPW = B // NW


def _make_pool_kernel():
    mesh = plsc.VectorSubcoreMesh(core_axis_name="c", subcore_axis_name="s")

    @functools.partial(
        pl.kernel,
        mesh=mesh,
        out_type=jax.ShapeDtypeStruct((B, EPAD), jnp.float32),
        compiler_params=pltpu.CompilerParams(
            use_tc_tiling_on_sc=False, needs_layout_passes=False
        ),
        scratch_types=[
            pltpu.VMEM((BPW, 2, HALF), jnp.int32),
            pltpu.VMEM((HALF, NW32), jnp.int32),
            pltpu.VMEM((HALF, NW32), jnp.int32),
            pltpu.VMEM((HALF, NW32), jnp.int32),
            pltpu.VMEM((HALF, NW32), jnp.int32),
            pltpu.VMEM((EPAD,), jnp.float32),
            pltpu.VMEM((EPAD,), jnp.float32),
            pltpu.SemaphoreType.DMA,
            pltpu.SemaphoreType.DMA,
            pltpu.SemaphoreType.DMA,
            pltpu.SemaphoreType.DMA,
            pltpu.SemaphoreType.DMA,
            pltpu.SemaphoreType.DMA,
        ],
    )
    def pool(idx_hbm, tbl_hbm, out_hbm, idx_v, ga0, ga1, gb0, gb1,
             orow_a, orow_b, sa0, sa1, sb0, sb1, soa, sob):
        wid = lax.axis_index("s") * NC + lax.axis_index("c")
        base = wid * BPW
        pltpu.sync_copy(idx_hbm.at[pl.ds(base, BPW)], idx_v)

        def fire(b, g0, g1, s0, s1):
            pltpu.async_copy(tbl_hbm.at[idx_v.at[b, 0]], g0, s0)
            pltpu.async_copy(tbl_hbm.at[idx_v.at[b, 1]], g1, s1)

        def drain(b, g0, g1, s0, s1):
            pltpu.make_async_copy(tbl_hbm.at[idx_v.at[b, 0]], g0, s0).wait()
            pltpu.make_async_copy(tbl_hbm.at[idx_v.at[b, 1]], g1, s1).wait()

        def sum_rows(gbuf, acc):
            def pair_body(i, a):
                r = 2 * i
                out = []
                for j in range(NBLK):
                    s = plsc.bitcast(
                        gbuf[r, pl.ds(16 * j, 16)], jnp.bfloat16
                    ) + plsc.bitcast(
                        gbuf[r + 1, pl.ds(16 * j, 16)], jnp.bfloat16
                    )
                    lo, hi = plsc.unpack(
                        s, format=plsc.PackFormat.INTERLEAVED
                    )
                    out.append(a[2 * j] + lo)
                    out.append(a[2 * j + 1] + hi)
                return tuple(out)
            return lax.fori_loop(0, HALF // 2, pair_body, acc, unroll=2)

        def process(b, g0, g1, s0, s1, orow, so):
            drain(b, g0, g1, s0, s1)
            acc = tuple(jnp.zeros((16,), jnp.float32) for _ in range(2 * NBLK))
            acc = sum_rows(g0, acc)
            acc = sum_rows(g1, acc)
            pltpu.make_async_copy(orow, out_hbm.at[base + b], so).wait()
            for j in range(NBLK):
                orow[pl.ds(16 * j, 16)] = acc[2 * j]
                orow[pl.ds(160 + 16 * j, 16)] = acc[2 * j + 1]
            pltpu.async_copy(orow, out_hbm.at[base + b], so)

        fire(0, ga0, ga1, sa0, sa1)
        pltpu.async_copy(orow_a, out_hbm.at[base], soa)
        pltpu.async_copy(orow_b, out_hbm.at[base + 1], sob)

        def pair_body(g, carry):
            b0 = 2 * g
            fire(b0 + 1, gb0, gb1, sb0, sb1)
            process(b0, ga0, ga1, sa0, sa1, orow_a, soa)
            b2 = jnp.minimum(b0 + 2, BPW - 1)
            fire(b2, ga0, ga1, sa0, sa1)
            process(b0 + 1, gb0, gb1, sb0, sb1, orow_b, sob)
            return carry

        lax.fori_loop(0, BPW // 2, pair_body, 0)

        drain(BPW - 1, ga0, ga1, sa0, sa1)
        pltpu.make_async_copy(orow_a, out_hbm.at[base + BPW - 2], soa).wait()
        pltpu.make_async_copy(orow_b, out_hbm.at[base + BPW - 1], sob).wait()

    return pool


_pool = _make_pool_kernel()

PADBLK = 5000
NHI = EMB - NW32


def _cvt_body(src_ref, dst_ref):
    u = jax.lax.bitcast_convert_type(src_ref[...], jnp.uint32)
    r = (u + 0x7FFF + ((u >> 16) & 1)) & jnp.uint32(0xFFFF0000)
    lo = r[:, :NW32] >> 16
    hi = r[:, NW32:]
    dst_ref[:, :NHI] = jax.lax.bitcast_convert_type(
        lo[:, :NHI] | hi, jnp.int32
    )
    dst_ref[:, NHI:] = jax.lax.bitcast_convert_type(lo[:, NHI:], jnp.int32)


def _cvt_table(tbl):
    return pl.pallas_call(
        _cvt_body,
        grid=(VOCAB // PADBLK,),
        in_specs=[pl.BlockSpec((PADBLK, EMB), lambda i: (i, 0))],
        out_specs=pl.BlockSpec((PADBLK, NW32), lambda i: (i, 0)),
        out_shape=jax.ShapeDtypeStruct((VOCAB, NW32), jnp.int32),
    )(tbl)


BLK = 512


def _mlp_body(enc_ref, tl_ref, w1_ref, b1_ref, w2_ref, b2_ref, out_ref):
    x = enc_ref[...] / tl_ref[...]
    h = jnp.dot(x, w1_ref[...], preferred_element_type=jnp.float32) + b1_ref[...]
    h = jnp.where(h > 0, h, jnp.exp(h) - 1.0)
    out_ref[...] = (
        jnp.dot(h, w2_ref[...], preferred_element_type=jnp.float32) + b2_ref[...]
    )


def kernel(input_text, text_len, emb_table, W1, b1, W2, b2):
    idx3 = input_text.astype(jnp.int32).reshape(B, 2, HALF)
    w1t = jnp.pad(W1, ((0, 0), (0, EPAD - EMB))).T
    w2t = W2.T
    b1r = b1.reshape(1, HID)
    b2r = b2.reshape(1, NCLS)
    tl2 = text_len.reshape(B, 1)

    tbl = _cvt_table(emb_table)
    encoded = _pool(idx3, tbl)

    logits = pl.pallas_call(
        _mlp_body,
        grid=(B // BLK,),
        in_specs=[
            pl.BlockSpec((BLK, EPAD), lambda i: (i, 0)),
            pl.BlockSpec((BLK, 1), lambda i: (i, 0)),
            pl.BlockSpec((EPAD, HID), lambda i: (0, 0)),
            pl.BlockSpec((1, HID), lambda i: (0, 0)),
            pl.BlockSpec((HID, NCLS), lambda i: (0, 0)),
            pl.BlockSpec((1, NCLS), lambda i: (0, 0)),
        ],
        out_specs=pl.BlockSpec((BLK, NCLS), lambda i: (i, 0)),
        out_shape=jax.ShapeDtypeStruct((B, NCLS), jnp.float32),
    )(encoded, tl2, w1t, b1r, w2t, b2r)
    return logits


# speedup vs baseline: 1.0634x; 1.0267x over previous
import functools

import jax
import jax.numpy as jnp
from jax import lax
from jax.experimental import pallas as pl
from jax.experimental.pallas import tpu as pltpu
from jax.experimental.pallas import tpu_sc as plsc

VOCAB = 100000
EMB = 300
EPAD = 320
NW32 = EPAD // 2
NBLK = EPAD // 32
HID = 100
NCLS = 1000
B = 4096
L = 200
S0 = 96
S1 = L - S0

NC, NS = 2, 16
NW = NC * NS
BPW = B // NW


def _make_pool_kernel():
    mesh = plsc.VectorSubcoreMesh(core_axis_name="c", subcore_axis_name="s")

    @functools.partial(
        pl.kernel,
        mesh=mesh,
        out_type=jax.ShapeDtypeStruct((B, EPAD), jnp.float32),
        compiler_params=pltpu.CompilerParams(
            use_tc_tiling_on_sc=False, needs_layout_passes=False
        ),
        scratch_types=[
            pltpu.VMEM((BPW, L), jnp.int32),
            pltpu.VMEM((S0, NW32), jnp.int32),
            pltpu.VMEM((S1, NW32), jnp.int32),
            pltpu.VMEM((S0, NW32), jnp.int32),
            pltpu.VMEM((S1, NW32), jnp.int32),
            pltpu.VMEM((EPAD,), jnp.float32),
            pltpu.VMEM((EPAD,), jnp.float32),
            pltpu.SemaphoreType.DMA,
            pltpu.SemaphoreType.DMA,
            pltpu.SemaphoreType.DMA,
            pltpu.SemaphoreType.DMA,
            pltpu.SemaphoreType.DMA,
            pltpu.SemaphoreType.DMA,
        ],
    )
    def pool(idx_hbm, tbl_hbm, out_hbm, idx_v, ga0, ga1, gb0, gb1,
             orow_a, orow_b, sa0, sa1, sb0, sb1, soa, sob):
        wid = lax.axis_index("s") * NC + lax.axis_index("c")
        base = wid * BPW
        pltpu.sync_copy(idx_hbm.at[pl.ds(base, BPW)], idx_v)

        def fire(b, g0, g1, s0, s1):
            pltpu.async_copy(tbl_hbm.at[idx_v.at[b, pl.ds(0, S0)]], g0, s0)
            pltpu.async_copy(tbl_hbm.at[idx_v.at[b, pl.ds(S0, S1)]], g1, s1)

        def drain(b, g0, g1, s0, s1):
            pltpu.make_async_copy(
                tbl_hbm.at[idx_v.at[b, pl.ds(0, S0)]], g0, s0
            ).wait()
            pltpu.make_async_copy(
                tbl_hbm.at[idx_v.at[b, pl.ds(S0, S1)]], g1, s1
            ).wait()

        def sum_rows(gbuf, nrows, acc):
            def pair_body(i, a):
                r = 2 * i
                out = []
                for j in range(NBLK):
                    s = plsc.bitcast(
                        gbuf[r, pl.ds(16 * j, 16)], jnp.bfloat16
                    ) + plsc.bitcast(
                        gbuf[r + 1, pl.ds(16 * j, 16)], jnp.bfloat16
                    )
                    lo, hi = plsc.unpack(
                        s, format=plsc.PackFormat.INTERLEAVED
                    )
                    out.append(a[2 * j] + lo)
                    out.append(a[2 * j + 1] + hi)
                return tuple(out)
            return lax.fori_loop(0, nrows // 2, pair_body, acc, unroll=2)

        def process(b, g0, g1, s0, s1, orow, so):
            drain(b, g0, g1, s0, s1)
            acc = tuple(jnp.zeros((16,), jnp.float32) for _ in range(2 * NBLK))
            acc = sum_rows(g0, S0, acc)
            acc = sum_rows(g1, S1, acc)
            pltpu.make_async_copy(orow, out_hbm.at[base + b], so).wait()
            for j in range(NBLK):
                orow[pl.ds(16 * j, 16)] = acc[2 * j]
                orow[pl.ds(160 + 16 * j, 16)] = acc[2 * j + 1]
            pltpu.async_copy(orow, out_hbm.at[base + b], so)

        fire(0, ga0, ga1, sa0, sa1)
        pltpu.async_copy(orow_a, out_hbm.at[base], soa)
        pltpu.async_copy(orow_b, out_hbm.at[base + 1], sob)

        def pair_body(g, carry):
            b0 = 2 * g
            fire(b0 + 1, gb0, gb1, sb0, sb1)
            process(b0, ga0, ga1, sa0, sa1, orow_a, soa)
            b2 = jnp.minimum(b0 + 2, BPW - 1)
            fire(b2, ga0, ga1, sa0, sa1)
            process(b0 + 1, gb0, gb1, sb0, sb1, orow_b, sob)
            return carry

        lax.fori_loop(0, BPW // 2, pair_body, 0)

        drain(BPW - 1, ga0, ga1, sa0, sa1)
        pltpu.make_async_copy(orow_a, out_hbm.at[base + BPW - 2], soa).wait()
        pltpu.make_async_copy(orow_b, out_hbm.at[base + BPW - 1], sob).wait()

    return pool


_pool = _make_pool_kernel()

PADBLK = 5000
NHI = EMB - NW32


def _cvt_body(src_ref, dst_ref):
    u = jax.lax.bitcast_convert_type(src_ref[...], jnp.uint32)
    r = (u + 0x7FFF + ((u >> 16) & 1)) & jnp.uint32(0xFFFF0000)
    lo = r[:, :NW32] >> 16
    hi = r[:, NW32:]
    dst_ref[:, :NHI] = jax.lax.bitcast_convert_type(
        lo[:, :NHI] | hi, jnp.int32
    )
    dst_ref[:, NHI:] = jax.lax.bitcast_convert_type(lo[:, NHI:], jnp.int32)


def _cvt_table(tbl):
    return pl.pallas_call(
        _cvt_body,
        grid=(VOCAB // PADBLK,),
        in_specs=[pl.BlockSpec((PADBLK, EMB), lambda i: (i, 0))],
        out_specs=pl.BlockSpec((PADBLK, NW32), lambda i: (i, 0)),
        out_shape=jax.ShapeDtypeStruct((VOCAB, NW32), jnp.int32),
    )(tbl)


BLK = 512


def _mlp_body(enc_ref, tl_ref, w1_ref, b1_ref, w2_ref, b2_ref, out_ref):
    x = enc_ref[...] / tl_ref[...]
    h = jnp.dot(x, w1_ref[...], preferred_element_type=jnp.float32) + b1_ref[...]
    h = jnp.where(h > 0, h, jnp.exp(h) - 1.0)
    out_ref[...] = (
        jnp.dot(h, w2_ref[...], preferred_element_type=jnp.float32) + b2_ref[...]
    )


def kernel(input_text, text_len, emb_table, W1, b1, W2, b2):
    idx2 = input_text.astype(jnp.int32)
    w1t = jnp.pad(W1, ((0, 0), (0, EPAD - EMB))).T
    w2t = W2.T
    b1r = b1.reshape(1, HID)
    b2r = b2.reshape(1, NCLS)
    tl2 = text_len.reshape(B, 1)

    tbl = _cvt_table(emb_table)
    encoded = _pool(idx2, tbl)

    logits = pl.pallas_call(
        _mlp_body,
        grid=(B // BLK,),
        in_specs=[
            pl.BlockSpec((BLK, EPAD), lambda i: (i, 0)),
            pl.BlockSpec((BLK, 1), lambda i: (i, 0)),
            pl.BlockSpec((EPAD, HID), lambda i: (0, 0)),
            pl.BlockSpec((1, HID), lambda i: (0, 0)),
            pl.BlockSpec((HID, NCLS), lambda i: (0, 0)),
            pl.BlockSpec((1, NCLS), lambda i: (0, 0)),
        ],
        out_specs=pl.BlockSpec((BLK, NCLS), lambda i: (i, 0)),
        out_shape=jax.ShapeDtypeStruct((B, NCLS), jnp.float32),
    )(encoded, tl2, w1t, b1r, w2t, b2r)
    return logits
